# Initial kernel scaffold; baseline (speedup 1.0000x reference)
import jax
import jax.numpy as jnp
from jax.experimental import pallas as pl


def kernel(x):
    raise NotImplementedError("write your pallas kernel here")


# speedup vs baseline: 11.5891x; 11.5891x over previous
import functools

import jax
import jax.numpy as jnp
from jax import lax
from jax.experimental import pallas as pl
from jax.experimental.pallas import tpu as pltpu
from jax.experimental.pallas import tpu_sc as plsc

KM = 8
LANES = 16
SPATIAL = 1024
ROWS_PER_CHUNK = 16

_SORT8 = [(0, 1), (2, 3), (4, 5), (6, 7), (0, 2), (1, 3), (4, 6), (5, 7),
          (1, 2), (5, 6), (0, 4), (1, 5), (2, 6), (3, 7), (2, 4), (3, 5),
          (1, 2), (3, 4), (5, 6)]
_BITONIC8 = [(0, 4), (1, 5), (2, 6), (3, 7), (0, 2), (1, 3), (4, 6), (5, 7),
             (0, 1), (2, 3), (4, 5), (6, 7)]


def _apply_network(v, net):
    v = list(v)
    for i, j in net:
        hi = jnp.maximum(v[i], v[j])
        lo = jnp.minimum(v[i], v[j])
        v[i], v[j] = hi, lo
    return v


def _merge_top8(a, b):
    c = [jnp.maximum(a[i], b[7 - i]) for i in range(KM)]
    return _apply_network(c, _BITONIC8)


def _make_sc_topk(B, CG):
    mesh = plsc.VectorSubcoreMesh(core_axis_name="c", subcore_axis_name="s")
    info = plsc.get_sparse_core_info()
    nc = info.num_cores

    @functools.partial(
        pl.kernel,
        out_type=jax.ShapeDtypeStruct((B, CG, KM, LANES), jnp.float32),
        mesh=mesh,
        scratch_types=[
            pltpu.VMEM((SPATIAL, LANES), jnp.float32),
            pltpu.VMEM((KM, LANES), jnp.float32),
        ],
        compiler_params=pltpu.CompilerParams(use_tc_tiling_on_sc=False),
    )
    def topk_kernel(x_hbm, out_hbm, slab, stage):
        b = lax.axis_index("s") * nc + lax.axis_index("c")

        def per_group(cg, carry):
            pltpu.sync_copy(x_hbm.at[b, :, cg, :], slab)

            def per_chunk(i, r):
                base = i * ROWS_PER_CHUNK
                rows = [slab[base + k, :] for k in range(ROWS_PER_CHUNK)]
                a = _apply_network(rows[:KM], _SORT8)
                bb = _apply_network(rows[KM:], _SORT8)
                c = _merge_top8(a, bb)
                return tuple(_merge_top8(list(r), c))

            neg_inf = jnp.full((LANES,), -jnp.inf, jnp.float32)
            r0 = (neg_inf,) * KM
            r = lax.fori_loop(0, SPATIAL // ROWS_PER_CHUNK, per_chunk, r0)
            for k in range(KM):
                stage[k, :] = r[k]
            pltpu.sync_copy(stage, out_hbm.at[b, cg])
            return carry

        lax.fori_loop(0, CG, per_group, 0)

    return topk_kernel


def kernel(x):
    B, H, W, C = x.shape
    CG = C // LANES
    xr = jnp.reshape(x, (B, H * W, CG, LANES))
    out = _make_sc_topk(B, CG)(xr)
    out = jnp.transpose(out, (0, 1, 3, 2))
    return jnp.reshape(out, (B, KM * C))
